# baseline (speedup 1.0000x reference)
import jax
import jax.numpy as jnp
from jax import lax
from jax.experimental import pallas as pl
from jax.experimental.pallas import tpu as pltpu

THRESH = 0.5
BLK_N = 8


def _bbox_block(m_ref, out_ref):
    m = m_ref[...]
    _, H, W = m.shape
    colmax = jnp.max(m, axis=1)
    rowmax = jnp.max(m, axis=2)
    wocc = colmax >= THRESH
    hocc = rowmax >= THRESH
    iw = lax.broadcasted_iota(jnp.int32, wocc.shape, 1)
    ih = lax.broadcasted_iota(jnp.int32, hocc.shape, 1)
    any_w = jnp.any(wocc, axis=1)
    any_h = jnp.any(hocc, axis=1)
    xmin = jnp.where(any_w, jnp.min(jnp.where(wocc, iw, W), axis=1), 0)
    xmax = jnp.where(any_w, jnp.max(jnp.where(wocc, iw, -1), axis=1) + 1, W)
    ymin = jnp.where(any_h, jnp.min(jnp.where(hocc, ih, H), axis=1), 0)
    ymax = jnp.where(any_h, jnp.max(jnp.where(hocc, ih, -1), axis=1) + 1, H)
    out_ref[...] = jnp.stack((ymin, xmin, ymax, xmax), axis=-1)


def kernel(mask):
    N, _, H, W = mask.shape
    m = mask.reshape(N, H, W)
    grid = (N // BLK_N,)
    return pl.pallas_call(
        _bbox_block,
        grid=grid,
        in_specs=[pl.BlockSpec((BLK_N, H, W), lambda i: (i, 0, 0))],
        out_specs=pl.BlockSpec((BLK_N, 4), lambda i: (i, 0)),
        out_shape=jax.ShapeDtypeStruct((N, 4), jnp.int32),
    )(m)
